# baseline (device time: 191584 ns/iter reference)
import jax
import jax.numpy as jnp
from jax import lax
from jax.experimental import pallas as pl
from jax.experimental.pallas import tpu as pltpu

N_DEV = 16
B, SQ, SKV, HQ_TOTAL, DH = 2, 512, 512, 128, 64
H_PER = HQ_TOTAL // N_DEV
HD = H_PER * DH
D_MODEL = 768
ROWS = B * SQ

_HALVES = [512, 256, 128, 64]
_RBUF_OFFS = [0, 512, 768, 896]


def _fused_attn_allreduce(q, k_ext, v_ext, wo):

    def body(q_ref, ke_ref, ve_ref, wo_ref, out_ref,
             kraw, vraw, acc_ref, rbuf_ref,
             ksem, vsem, rs_send, rs_recv, ag_send, ag_recv):
        my = lax.axis_index("i")
        j = lax.rem(my, 4)
        z = lax.div(my, 4)
        bits = [
            jnp.where((j == 1) | (j == 2), 1, 0),
            jnp.where(j >= 2, 1, 0),
            lax.rem(z, 2),
            lax.div(z, 2),
        ]
        partners = [my ^ 1, my ^ 3, my ^ 4, my ^ 8]

        barrier_sem = pltpu.get_barrier_semaphore()
        for p in partners:
            pl.semaphore_signal(
                barrier_sem, inc=1,
                device_id=(p,), device_id_type=pl.DeviceIdType.MESH,
            )

        kcp, vcp = [], []
        for b in range(B):
            kc = pltpu.make_async_copy(
                ke_ref.at[b, :, pl.ds(my * HD, HD)], kraw.at[b],
                ksem.at[b],
            )
            vc = pltpu.make_async_copy(
                ve_ref.at[b, :, pl.ds(my * HD, HD)], vraw.at[b],
                vsem.at[b],
            )
            kc.start()
            vc.start()
            kcp.append(kc)
            vcp.append(vc)

        qi = lax.broadcasted_iota(jnp.int32, (SQ, SKV), 0)
        ki = lax.broadcasted_iota(jnp.int32, (SQ, SKV), 1)
        mask = (jnp.abs(qi - ki) <= 128) | (ki < 32) | (qi < 32)

        def compute_batch(b):
            kcp[b].wait()
            vcp[b].wait()
            kb = kraw[b].astype(jnp.bfloat16)
            vb = vraw[b].astype(jnp.bfloat16)
            for h in range(H_PER):
                qh = q_ref[b, h, :, :]
                kh = kb[:, h * DH:(h + 1) * DH]
                vh = vb[:, h * DH:(h + 1) * DH]
                s = lax.dot_general(
                    qh, kh, (((1,), (1,)), ((), ())),
                    preferred_element_type=jnp.float32,
                ) * 0.125
                s = jnp.where(mask, s, -1e9)
                e = jnp.exp(s)
                w = (e / jnp.sum(e, axis=1, keepdims=True)).astype(
                    jnp.bfloat16
                )
                ctx = lax.dot_general(
                    w, vh, (((1,), (0,)), ((), ())),
                    preferred_element_type=jnp.float32,
                ).astype(jnp.bfloat16)
                pw = lax.dot_general(
                    ctx, wo_ref[h * DH:(h + 1) * DH, :],
                    (((1,), (0,)), ((), ())),
                    preferred_element_type=jnp.float32,
                )
                if h == 0:
                    acc_ref[b * SQ:(b + 1) * SQ, :] = pw
                else:
                    acc_ref[b * SQ:(b + 1) * SQ, :] = (
                        acc_ref[b * SQ:(b + 1) * SQ, :] + pw
                    )
            out_ref[b * SQ:(b + 1) * SQ, :] = (
                acc_ref[b * SQ:(b + 1) * SQ, :].astype(jnp.bfloat16)
            )

        b1 = bits[0]

        @pl.when(b1 == 0)
        def _():
            compute_batch(1)

        @pl.when(b1 == 1)
        def _():
            compute_batch(0)

        pl.semaphore_wait(barrier_sem, len(partners))

        h1 = _HALVES[0]
        send_off1 = (1 - b1) * h1
        rdma1 = pltpu.make_async_remote_copy(
            src_ref=out_ref.at[pl.ds(send_off1, h1)],
            dst_ref=rbuf_ref.at[pl.ds(_RBUF_OFFS[0], h1)],
            send_sem=rs_send.at[0],
            recv_sem=rs_recv.at[0],
            device_id=(partners[0],),
            device_id_type=pl.DeviceIdType.MESH,
        )
        rdma1.start()

        @pl.when(b1 == 0)
        def _():
            compute_batch(0)

        @pl.when(b1 == 1)
        def _():
            compute_batch(1)

        S = jnp.int32(0)
        for k in range(4):
            h = _HALVES[k]
            if k == 0:
                keep_off = S + bits[k] * h
                rdma1.wait()
                out_ref[pl.ds(keep_off, h), :] = (
                    out_ref[pl.ds(keep_off, h), :]
                    + rbuf_ref[pl.ds(_RBUF_OFFS[k], h), :]
                )
                S = keep_off
                continue
            send_off = S + (1 - bits[k]) * h
            keep_off = S + bits[k] * h
            rdma = pltpu.make_async_remote_copy(
                src_ref=out_ref.at[pl.ds(send_off, h)],
                dst_ref=rbuf_ref.at[pl.ds(_RBUF_OFFS[k], h)],
                send_sem=rs_send.at[k],
                recv_sem=rs_recv.at[k],
                device_id=(partners[k],),
                device_id_type=pl.DeviceIdType.MESH,
            )
            rdma.start()
            rdma.wait()
            out_ref[pl.ds(keep_off, h), :] = (
                out_ref[pl.ds(keep_off, h), :]
                + rbuf_ref[pl.ds(_RBUF_OFFS[k], h), :]
            )
            S = keep_off

        for k in (3, 2, 1, 0):
            g = _HALVES[k]
            rdma = pltpu.make_async_remote_copy(
                src_ref=out_ref.at[pl.ds(S, g)],
                dst_ref=out_ref.at[pl.ds(S, g)],
                send_sem=ag_send.at[k],
                recv_sem=ag_recv.at[k],
                device_id=(partners[k],),
                device_id_type=pl.DeviceIdType.MESH,
            )
            rdma.start()
            rdma.wait()
            S = S - bits[k] * g

    return pl.pallas_call(
        body,
        out_shape=jax.ShapeDtypeStruct((ROWS, D_MODEL), jnp.bfloat16),
        in_specs=[
            pl.BlockSpec(memory_space=pltpu.VMEM),
            pl.BlockSpec(memory_space=pltpu.MemorySpace.HBM),
            pl.BlockSpec(memory_space=pltpu.MemorySpace.HBM),
            pl.BlockSpec(memory_space=pltpu.VMEM),
        ],
        out_specs=pl.BlockSpec(memory_space=pltpu.VMEM),
        scratch_shapes=[
            pltpu.VMEM((B, SKV, HD), jnp.float32),
            pltpu.VMEM((B, SKV, HD), jnp.float32),
            pltpu.VMEM((ROWS, D_MODEL), jnp.float32),
            pltpu.VMEM((ROWS, D_MODEL), jnp.bfloat16),
            pltpu.SemaphoreType.DMA((B,)),
            pltpu.SemaphoreType.DMA((B,)),
            pltpu.SemaphoreType.DMA((4,)),
            pltpu.SemaphoreType.DMA((4,)),
            pltpu.SemaphoreType.DMA((4,)),
            pltpu.SemaphoreType.DMA((4,)),
        ],
        compiler_params=pltpu.CompilerParams(collective_id=0),
    )(q, k_ext, v_ext, wo)


def kernel(x, Wq, K_ext, V_ext, Wo):
    xb = x.astype(jnp.bfloat16)
    Qt = jnp.einsum(
        "bsd,dhf->bhsf",
        xb,
        Wq.astype(jnp.bfloat16).reshape(D_MODEL, H_PER, DH),
        preferred_element_type=jnp.float32,
    ).astype(jnp.bfloat16)

    out = _fused_attn_allreduce(
        Qt,
        K_ext.reshape(B, SKV, HQ_TOTAL * DH),
        V_ext.reshape(B, SKV, HQ_TOTAL * DH),
        Wo.astype(jnp.bfloat16),
    )
    return out.reshape(B, SQ, D_MODEL)


# device time: 125265 ns/iter; 1.5294x vs baseline; 1.5294x over previous
import jax
import jax.numpy as jnp
from jax import lax
from jax.experimental import pallas as pl
from jax.experimental.pallas import tpu as pltpu

N_DEV = 16
B, SQ, SKV, HQ_TOTAL, DH = 2, 512, 512, 128, 64
H_PER = HQ_TOTAL // N_DEV
D_MODEL = 768
ROWS = B * SQ

_HALVES = [512, 256, 128, 64]
_RBUF_OFFS = [0, 512, 768, 896]


def _fused_attn_allreduce(q, k, v, wo):

    def body(q_ref, k_ref, v_ref, wo_ref, out_ref, acc_ref, rbuf_ref,
             rs_send, rs_recv, ag_send, ag_recv):
        my = lax.axis_index("i")
        j = lax.rem(my, 4)
        z = lax.div(my, 4)
        bits = [
            jnp.where((j == 1) | (j == 2), 1, 0),
            jnp.where(j >= 2, 1, 0),
            lax.rem(z, 2),
            lax.div(z, 2),
        ]
        partners = [my ^ 1, my ^ 3, my ^ 4, my ^ 8]

        barrier_sem = pltpu.get_barrier_semaphore()
        for p in partners:
            pl.semaphore_signal(
                barrier_sem, inc=1,
                device_id=(p,), device_id_type=pl.DeviceIdType.MESH,
            )

        qi = lax.broadcasted_iota(jnp.int32, (SQ, SKV), 0)
        ki = lax.broadcasted_iota(jnp.int32, (SQ, SKV), 1)
        mask = (jnp.abs(qi - ki) <= 128) | (ki < 32) | (qi < 32)

        def compute_batch(b):
            for h in range(H_PER):
                qh = q_ref[b, h, :, :]
                kh = k_ref[b, h, :, :]
                vh = v_ref[b, h, :, :]
                s = lax.dot_general(
                    qh, kh, (((1,), (1,)), ((), ())),
                    preferred_element_type=jnp.float32,
                ) * 0.125
                s = jnp.where(mask, s, -1e9)
                e = jnp.exp(s)
                w = (e / jnp.sum(e, axis=1, keepdims=True)).astype(
                    jnp.bfloat16
                )
                ctx = lax.dot_general(
                    w, vh, (((1,), (0,)), ((), ())),
                    preferred_element_type=jnp.float32,
                ).astype(jnp.bfloat16)
                pw = lax.dot_general(
                    ctx, wo_ref[h * DH:(h + 1) * DH, :],
                    (((1,), (0,)), ((), ())),
                    preferred_element_type=jnp.float32,
                )
                if h == 0:
                    acc_ref[b * SQ:(b + 1) * SQ, :] = pw
                else:
                    acc_ref[b * SQ:(b + 1) * SQ, :] = (
                        acc_ref[b * SQ:(b + 1) * SQ, :] + pw
                    )
            out_ref[b * SQ:(b + 1) * SQ, :] = (
                acc_ref[b * SQ:(b + 1) * SQ, :].astype(jnp.bfloat16)
            )

        b1 = bits[0]

        @pl.when(b1 == 0)
        def _():
            compute_batch(1)

        @pl.when(b1 == 1)
        def _():
            compute_batch(0)

        pl.semaphore_wait(barrier_sem, len(partners))

        h1 = _HALVES[0]
        send_off1 = (1 - b1) * h1
        rdma1 = pltpu.make_async_remote_copy(
            src_ref=out_ref.at[pl.ds(send_off1, h1)],
            dst_ref=rbuf_ref.at[pl.ds(_RBUF_OFFS[0], h1)],
            send_sem=rs_send.at[0],
            recv_sem=rs_recv.at[0],
            device_id=(partners[0],),
            device_id_type=pl.DeviceIdType.MESH,
        )
        rdma1.start()

        @pl.when(b1 == 0)
        def _():
            compute_batch(0)

        @pl.when(b1 == 1)
        def _():
            compute_batch(1)

        S = jnp.int32(0)
        for k in range(4):
            h = _HALVES[k]
            if k == 0:
                keep_off = S + bits[k] * h
                rdma1.wait()
                out_ref[pl.ds(keep_off, h), :] = (
                    out_ref[pl.ds(keep_off, h), :]
                    + rbuf_ref[pl.ds(_RBUF_OFFS[k], h), :]
                )
                S = keep_off
                continue
            send_off = S + (1 - bits[k]) * h
            keep_off = S + bits[k] * h
            rdma = pltpu.make_async_remote_copy(
                src_ref=out_ref.at[pl.ds(send_off, h)],
                dst_ref=rbuf_ref.at[pl.ds(_RBUF_OFFS[k], h)],
                send_sem=rs_send.at[k],
                recv_sem=rs_recv.at[k],
                device_id=(partners[k],),
                device_id_type=pl.DeviceIdType.MESH,
            )
            rdma.start()
            rdma.wait()
            out_ref[pl.ds(keep_off, h), :] = (
                out_ref[pl.ds(keep_off, h), :]
                + rbuf_ref[pl.ds(_RBUF_OFFS[k], h), :]
            )
            S = keep_off

        for k in (3, 2, 1, 0):
            g = _HALVES[k]
            rdma = pltpu.make_async_remote_copy(
                src_ref=out_ref.at[pl.ds(S, g)],
                dst_ref=out_ref.at[pl.ds(S, g)],
                send_sem=ag_send.at[k],
                recv_sem=ag_recv.at[k],
                device_id=(partners[k],),
                device_id_type=pl.DeviceIdType.MESH,
            )
            rdma.start()
            rdma.wait()
            S = S - bits[k] * g

    return pl.pallas_call(
        body,
        out_shape=jax.ShapeDtypeStruct((ROWS, D_MODEL), jnp.bfloat16),
        in_specs=[pl.BlockSpec(memory_space=pltpu.VMEM)] * 4,
        out_specs=pl.BlockSpec(memory_space=pltpu.VMEM),
        scratch_shapes=[
            pltpu.VMEM((ROWS, D_MODEL), jnp.float32),
            pltpu.VMEM((ROWS, D_MODEL), jnp.bfloat16),
            pltpu.SemaphoreType.DMA((4,)),
            pltpu.SemaphoreType.DMA((4,)),
            pltpu.SemaphoreType.DMA((4,)),
            pltpu.SemaphoreType.DMA((4,)),
        ],
        compiler_params=pltpu.CompilerParams(collective_id=0),
    )(q, k, v, wo)


def kernel(x, Wq, K_ext, V_ext, Wo):
    my = lax.axis_index("i")

    xb = x.astype(jnp.bfloat16)
    Qt = jnp.einsum(
        "bsd,dhf->bhsf",
        xb,
        Wq.astype(jnp.bfloat16).reshape(D_MODEL, H_PER, DH),
        preferred_element_type=jnp.float32,
    ).astype(jnp.bfloat16)

    K = lax.dynamic_slice_in_dim(K_ext, my * H_PER, H_PER, axis=2)
    V = lax.dynamic_slice_in_dim(V_ext, my * H_PER, H_PER, axis=2)
    eye = jnp.eye(DH, dtype=jnp.bfloat16)
    Kt = jnp.einsum(
        "bshd,de->bhse", K.astype(jnp.bfloat16), eye,
        preferred_element_type=jnp.float32,
    ).astype(jnp.bfloat16)
    Vt = jnp.einsum(
        "bshd,de->bhse", V.astype(jnp.bfloat16), eye,
        preferred_element_type=jnp.float32,
    ).astype(jnp.bfloat16)

    out = _fused_attn_allreduce(Qt, Kt, Vt, Wo.astype(jnp.bfloat16))
    return out.reshape(B, SQ, D_MODEL)


# device time: 104078 ns/iter; 1.8408x vs baseline; 1.2036x over previous
import jax
import jax.numpy as jnp
from jax import lax
from jax.experimental import pallas as pl
from jax.experimental.pallas import tpu as pltpu

N_DEV = 16
B, SQ, SKV, HQ_TOTAL, DH = 2, 512, 512, 128, 64
H_PER = HQ_TOTAL // N_DEV
D_MODEL = 768
ROWS = B * SQ

_HALVES = [512, 256, 128, 64]
_RBUF_OFFS = [0, 512, 768, 896]
CW = D_MODEL // 2


def _fused_attn_allreduce(q, k, v, wo):

    def body(q_ref, k_ref, v_ref, wo_ref, out_ref, acc_ref, rbuf_ref,
             rs_send, rs_recv, ag_send, ag_recv):
        my = lax.axis_index("i")
        j = lax.rem(my, 4)
        z = lax.div(my, 4)
        bits = [
            jnp.where((j == 1) | (j == 2), 1, 0),
            jnp.where(j >= 2, 1, 0),
            lax.rem(z, 2),
            lax.div(z, 2),
        ]
        partners = [my ^ 1, my ^ 3, my ^ 4, my ^ 8]

        barrier_sem = pltpu.get_barrier_semaphore()
        for p in partners:
            pl.semaphore_signal(
                barrier_sem, inc=1,
                device_id=(p,), device_id_type=pl.DeviceIdType.MESH,
            )

        qi = lax.broadcasted_iota(jnp.int32, (SQ, SKV), 0)
        ki = lax.broadcasted_iota(jnp.int32, (SQ, SKV), 1)
        mask = (jnp.abs(qi - ki) <= 128) | (ki < 32) | (qi < 32)

        def compute_batch(b):
            for h in range(H_PER):
                qh = q_ref[b, h, :, :]
                kh = k_ref[b, h, :, :]
                vh = v_ref[b, h, :, :]
                s = lax.dot_general(
                    qh, kh, (((1,), (1,)), ((), ())),
                    preferred_element_type=jnp.float32,
                ) * 0.125
                s = jnp.where(mask, s, -1e9)
                e = jnp.exp(s)
                w = (e / jnp.sum(e, axis=1, keepdims=True)).astype(
                    jnp.bfloat16
                )
                ctx = lax.dot_general(
                    w, vh, (((1,), (0,)), ((), ())),
                    preferred_element_type=jnp.float32,
                ).astype(jnp.bfloat16)
                pw = lax.dot_general(
                    ctx, wo_ref[h * DH:(h + 1) * DH, :],
                    (((1,), (0,)), ((), ())),
                    preferred_element_type=jnp.float32,
                )
                if h == 0:
                    acc_ref[b * SQ:(b + 1) * SQ, :] = pw
                else:
                    acc_ref[b * SQ:(b + 1) * SQ, :] = (
                        acc_ref[b * SQ:(b + 1) * SQ, :] + pw
                    )
            out_ref[b * SQ:(b + 1) * SQ, :] = (
                acc_ref[b * SQ:(b + 1) * SQ, :].astype(jnp.bfloat16)
            )

        b1 = bits[0]

        @pl.when(b1 == 0)
        def _():
            compute_batch(1)

        @pl.when(b1 == 1)
        def _():
            compute_batch(0)

        pl.semaphore_wait(barrier_sem, len(partners))

        h1 = _HALVES[0]
        rdma1 = pltpu.make_async_remote_copy(
            src_ref=out_ref.at[pl.ds((1 - b1) * h1, h1), pl.ds(0, CW)],
            dst_ref=rbuf_ref.at[pl.ds(_RBUF_OFFS[0], h1), pl.ds(0, CW)],
            send_sem=rs_send.at[0],
            recv_sem=rs_recv.at[0],
            device_id=(partners[0],),
            device_id_type=pl.DeviceIdType.MESH,
        )
        rdma1.start()

        @pl.when(b1 == 0)
        def _():
            compute_batch(0)

        @pl.when(b1 == 1)
        def _():
            compute_batch(1)

        ORDER_A = (0, 1, 2, 3)
        ORDER_B = (2, 3, 0, 1)

        def rs_start(order, k, S, c0, so):
            idx = order[k]
            h = _HALVES[k]
            rdma = pltpu.make_async_remote_copy(
                src_ref=out_ref.at[pl.ds(S + (1 - bits[idx]) * h, h),
                                   pl.ds(c0, CW)],
                dst_ref=rbuf_ref.at[pl.ds(_RBUF_OFFS[k], h),
                                    pl.ds(c0, CW)],
                send_sem=rs_send.at[so + k],
                recv_sem=rs_recv.at[so + k],
                device_id=(partners[idx],),
                device_id_type=pl.DeviceIdType.MESH,
            )
            rdma.start()
            return rdma, S + bits[idx] * h

        def rs_finish(rdma, k, keep_off, c0):
            rdma.wait()
            h = _HALVES[k]
            out_ref[pl.ds(keep_off, h), pl.ds(c0, CW)] = (
                out_ref[pl.ds(keep_off, h), pl.ds(c0, CW)]
                + rbuf_ref[pl.ds(_RBUF_OFFS[k], h), pl.ds(c0, CW)]
            )

        def ag_start(order, k, S, c0, so):
            g = _HALVES[k]
            rdma = pltpu.make_async_remote_copy(
                src_ref=out_ref.at[pl.ds(S, g), pl.ds(c0, CW)],
                dst_ref=out_ref.at[pl.ds(S, g), pl.ds(c0, CW)],
                send_sem=ag_send.at[so + k],
                recv_sem=ag_recv.at[so + k],
                device_id=(partners[order[k]],),
                device_id_type=pl.DeviceIdType.MESH,
            )
            rdma.start()
            return rdma

        SA = jnp.int32(0)
        SB = jnp.int32(0)
        keepA = SA + bits[0] * _HALVES[0]
        rB, keepB = rs_start(ORDER_B, 0, SB, CW, 4)
        rs_finish(rdma1, 0, keepA, 0)
        SA = keepA
        rA, keepA = rs_start(ORDER_A, 1, SA, 0, 0)
        rs_finish(rB, 0, keepB, CW)
        SB = keepB
        rB, keepB = rs_start(ORDER_B, 1, SB, CW, 4)
        rs_finish(rA, 1, keepA, 0)
        SA = keepA
        rA, keepA = rs_start(ORDER_A, 2, SA, 0, 0)
        rs_finish(rB, 1, keepB, CW)
        SB = keepB
        rB, keepB = rs_start(ORDER_B, 2, SB, CW, 4)
        rs_finish(rA, 2, keepA, 0)
        SA = keepA
        rA, keepA = rs_start(ORDER_A, 3, SA, 0, 0)
        rs_finish(rB, 2, keepB, CW)
        SB = keepB
        rB, keepB = rs_start(ORDER_B, 3, SB, CW, 4)
        rs_finish(rA, 3, keepA, 0)
        SA = keepA
        gA = ag_start(ORDER_A, 3, SA, 0, 0)
        rs_finish(rB, 3, keepB, CW)
        SB = keepB
        gB = ag_start(ORDER_B, 3, SB, CW, 4)
        for k in (2, 1, 0):
            gA.wait()
            SA = SA - bits[ORDER_A[k + 1]] * _HALVES[k + 1]
            gA = ag_start(ORDER_A, k, SA, 0, 0)
            gB.wait()
            SB = SB - bits[ORDER_B[k + 1]] * _HALVES[k + 1]
            gB = ag_start(ORDER_B, k, SB, CW, 4)
        gA.wait()
        gB.wait()

    return pl.pallas_call(
        body,
        out_shape=jax.ShapeDtypeStruct((ROWS, D_MODEL), jnp.bfloat16),
        in_specs=[pl.BlockSpec(memory_space=pltpu.VMEM)] * 4,
        out_specs=pl.BlockSpec(memory_space=pltpu.VMEM),
        scratch_shapes=[
            pltpu.VMEM((ROWS, D_MODEL), jnp.float32),
            pltpu.VMEM((ROWS, D_MODEL), jnp.bfloat16),
            pltpu.SemaphoreType.DMA((8,)),
            pltpu.SemaphoreType.DMA((8,)),
            pltpu.SemaphoreType.DMA((8,)),
            pltpu.SemaphoreType.DMA((8,)),
        ],
        compiler_params=pltpu.CompilerParams(collective_id=0),
    )(q, k, v, wo)


def kernel(x, Wq, K_ext, V_ext, Wo):
    my = lax.axis_index("i")

    xb = x.astype(jnp.bfloat16)
    Q = jnp.einsum(
        "bsd,df->bsf", xb, Wq.astype(jnp.bfloat16),
        preferred_element_type=jnp.float32,
    ).astype(jnp.bfloat16).reshape(B, SQ, H_PER, DH)
    Qt = jnp.transpose(Q, (0, 2, 1, 3))

    K = lax.dynamic_slice_in_dim(K_ext, my * H_PER, H_PER, axis=2)
    V = lax.dynamic_slice_in_dim(V_ext, my * H_PER, H_PER, axis=2)
    Kt = jnp.transpose(K.astype(jnp.bfloat16), (0, 2, 1, 3))
    Vt = jnp.transpose(V.astype(jnp.bfloat16), (0, 2, 1, 3))

    out = _fused_attn_allreduce(Qt, Kt, Vt, Wo.astype(jnp.bfloat16))
    return out.reshape(B, SQ, D_MODEL)
